# baseline (speedup 1.0000x reference)
import jax
import jax.numpy as jnp
from jax.experimental import pallas as pl
from jax.experimental.pallas import tpu as pltpu

_TOKEN_BLOCK = 512
_NBUF = 4


def _router_body(x_hbm, w_ref, b_ref, o_ref, buf, sems):
    tokens = o_ref.shape[0]
    blk = _TOKEN_BLOCK
    nsteps = tokens // blk

    def copy_in(step, slot):
        return pltpu.make_async_copy(
            x_hbm.at[pl.ds(step * blk, blk), :], buf.at[slot], sems.at[slot])

    for s in range(min(_NBUF, nsteps)):
        copy_in(s, s).start()

    def step_fn(i, carry):
        slot = jax.lax.rem(i, _NBUF)
        copy_in(i, slot).wait()
        o_ref[pl.ds(i * blk, blk), :] = jax.lax.dot_general(
            buf[slot], w_ref[...],
            dimension_numbers=(((1,), (1,)), ((), ())),
            preferred_element_type=jnp.float32,
        ) + b_ref[...]
        nxt = i + _NBUF

        @pl.when(nxt < nsteps)
        def _():
            copy_in(nxt, slot).start()

        return carry

    jax.lax.fori_loop(0, nsteps, step_fn, 0)


def kernel(x, W, b):
    tokens, d = x.shape
    n_experts = W.shape[0]
    return pl.pallas_call(
        _router_body,
        in_specs=[
            pl.BlockSpec(memory_space=pltpu.MemorySpace.HBM),
            pl.BlockSpec(memory_space=pltpu.MemorySpace.VMEM),
            pl.BlockSpec(memory_space=pltpu.MemorySpace.VMEM),
        ],
        out_specs=pl.BlockSpec(memory_space=pltpu.MemorySpace.VMEM),
        out_shape=jax.ShapeDtypeStruct((tokens, n_experts), jnp.float32),
        scratch_shapes=[
            pltpu.MemorySpace.VMEM((_NBUF, _TOKEN_BLOCK, d), jnp.float32),
            pltpu.SemaphoreType.DMA((_NBUF,)),
        ],
    )(x, W, b.reshape(1, n_experts))


# speedup vs baseline: 1.0685x; 1.0685x over previous
import jax
import jax.numpy as jnp
from jax.experimental import pallas as pl

_TOKEN_BLOCK = 512


def _router_body(xa_ref, xb_ref, wa_ref, wb_ref, b_ref, o_ref):
    dn = (((1,), (1,)), ((), ()))
    acc = jax.lax.dot_general(xa_ref[...], wa_ref[...], dimension_numbers=dn,
                              preferred_element_type=jnp.float32)
    acc = acc + jax.lax.dot_general(xb_ref[...], wb_ref[...],
                                    dimension_numbers=dn,
                                    preferred_element_type=jnp.float32)
    o_ref[...] = acc + b_ref[...]


def kernel(x, W, b):
    tokens, d = x.shape
    n_experts = W.shape[0]
    blk = _TOKEN_BLOCK
    hk = d // 2
    return pl.pallas_call(
        _router_body,
        grid=(tokens // blk,),
        in_specs=[
            pl.BlockSpec((blk, hk), lambda i: (i, 0)),
            pl.BlockSpec((blk, hk), lambda i: (i, 1)),
            pl.BlockSpec((n_experts, hk), lambda i: (0, 0)),
            pl.BlockSpec((n_experts, hk), lambda i: (0, 1)),
            pl.BlockSpec((1, n_experts), lambda i: (0, 0)),
        ],
        out_specs=pl.BlockSpec((blk, n_experts), lambda i: (i, 0)),
        out_shape=jax.ShapeDtypeStruct((tokens, n_experts), jnp.float32),
    )(x, x, W, W, b.reshape(1, n_experts))
